# Initial kernel scaffold; baseline (speedup 1.0000x reference)
import jax
import jax.numpy as jnp
from jax.experimental import pallas as pl


def kernel(node_features, edge_features, edge_index, Wq_n, Wk_n, Wv_n, Wo_n, Wq_e, Wk_e, Wv_e, Wo_e, W_node_score, W_edge_score, W_cls1, b_cls1, W_cls2, b_cls2):
    raise NotImplementedError("write your pallas kernel here")


# speedup vs baseline: 2.4180x; 2.4180x over previous
import math

import jax
import jax.numpy as jnp
from jax import lax
from jax.experimental import pallas as pl

_B, _E, _D = 8, 1024, 128
_H = 4
_DH = _D // _H
_NC = 16
_K = _E // 2
_HI = lax.Precision.HIGHEST
_DEF = lax.Precision.DEFAULT


def _edge_body(srcT_ref, dstT_ref, src_ref, dst_ref, x_ref,
               wq_ref, wk_ref, wv_ref, wo_ref, wes_ref,
               wc1_ref, bc1_ref, wc2_ref, bc2_ref, out_ref):
    x = x_ref[0]
    src = src_ref[0]
    dst = dst_ref[0]
    srcT = srcT_ref[0]
    dstT = dstT_ref[0]

    q = jnp.dot(x, wq_ref[...], precision=_DEF, preferred_element_type=jnp.float32)
    k = jnp.dot(x, wk_ref[...], precision=_DEF, preferred_element_type=jnp.float32)
    v = jnp.dot(x, wv_ref[...], precision=_DEF, preferred_element_type=jnp.float32)

    rows = lax.broadcasted_iota(jnp.int32, (_E, _E), 0)
    cols = lax.broadcasted_iota(jnp.int32, (_E, _E), 1)
    diag = rows == cols
    adj = ((srcT == src) | (srcT == dst) | (dstT == src) | (dstT == dst) | diag)

    neg = jnp.float32(-1e9)
    scale = jnp.float32(math.sqrt(_DH))
    colsum = jnp.zeros((1, _E), jnp.float32)
    ohs = []
    for h in range(_H):
        lo, hi = h * _DH, (h + 1) * _DH
        qh = q[:, lo:hi]
        kh = k[:, lo:hi]
        vh = v[:, lo:hi]
        s = lax.dot_general(qh, kh, (((1,), (1,)), ((), ())),
                            precision=_DEF, preferred_element_type=jnp.float32)
        s = jnp.where(adj, s / scale, neg)
        m = jnp.max(s, axis=1, keepdims=True)
        p = jnp.exp(s - m)
        z = jnp.sum(p, axis=1, keepdims=True)
        a = p / z
        colsum = colsum + jnp.sum(a, axis=0, keepdims=True)
        oh = jnp.dot(p, vh, precision=_DEF, preferred_element_type=jnp.float32) / z
        ohs.append(oh)

    eo = jnp.dot(jnp.concatenate(ohs, axis=1), wo_ref[...], precision=_DEF,
                 preferred_element_type=jnp.float32)
    lin = jnp.dot(eo, wes_ref[...], precision=_DEF,
                  preferred_element_type=jnp.float32)
    ident = diag.astype(jnp.float32)
    linT = lax.dot_general(lin, ident, (((0,), (0,)), ((), ())),
                           precision=_HI, preferred_element_type=jnp.float32)
    s_col = colsum * jnp.float32(1.0 / (_H * _E)) + linT
    s_row = lax.dot_general(ident, s_col, (((1,), (1,)), ((), ())),
                            precision=_HI, preferred_element_type=jnp.float32)

    beats = ((s_col > s_row) | ((s_col == s_row) & (cols < rows))) & (~diag)
    rank = jnp.sum(beats.astype(jnp.float32), axis=1, keepdims=True)
    keep = (rank < jnp.float32(_K)).astype(jnp.float32)

    pruned = eo * keep
    h1 = (jnp.dot(pruned, wc1_ref[...], precision=_DEF,
                  preferred_element_type=jnp.float32) + bc1_ref[...])
    g = jnp.float32(0.5) * h1 * (jnp.float32(1.0)
                                 + lax.erf(h1 * jnp.float32(1.0 / math.sqrt(2.0))))
    out_ref[0] = (jnp.dot(g, wc2_ref[...], precision=_DEF,
                          preferred_element_type=jnp.float32) + bc2_ref[...])


def kernel(node_features, edge_features, edge_index, Wq_n, Wk_n, Wv_n, Wo_n,
           Wq_e, Wk_e, Wv_e, Wo_e, W_node_score, W_edge_score,
           W_cls1, b_cls1, W_cls2, b_cls2):
    src = edge_index[:, 0, :].astype(jnp.float32)
    dst = edge_index[:, 1, :].astype(jnp.float32)
    srcT = src[:, :, None]
    dstT = dst[:, :, None]
    srcR = src[:, None, :]
    dstR = dst[:, None, :]
    bc1 = b_cls1.reshape(1, _D)
    bc2 = b_cls2.reshape(1, _NC)

    full = lambda shape: pl.BlockSpec(shape, lambda b: tuple(0 for _ in shape))
    out = pl.pallas_call(
        _edge_body,
        grid=(_B,),
        in_specs=[
            pl.BlockSpec((1, _E, 1), lambda b: (b, 0, 0)),
            pl.BlockSpec((1, _E, 1), lambda b: (b, 0, 0)),
            pl.BlockSpec((1, 1, _E), lambda b: (b, 0, 0)),
            pl.BlockSpec((1, 1, _E), lambda b: (b, 0, 0)),
            pl.BlockSpec((1, _E, _D), lambda b: (b, 0, 0)),
            full((_D, _D)),
            full((_D, _D)),
            full((_D, _D)),
            full((_D, _D)),
            full((_D, 1)),
            full((_D, _D)),
            full((1, _D)),
            full((_D, _NC)),
            full((1, _NC)),
        ],
        out_specs=pl.BlockSpec((1, _E, _NC), lambda b: (b, 0, 0)),
        out_shape=jax.ShapeDtypeStruct((_B, _E, _NC), jnp.float32),
    )(srcT, dstT, srcR, dstR, edge_features, Wq_e, Wk_e, Wv_e, Wo_e,
      W_edge_score, W_cls1, bc1, W_cls2, bc2)
    return out
